# Initial kernel scaffold; baseline (speedup 1.0000x reference)
import functools

import jax
import jax.numpy as jnp
from jax import lax
from jax.experimental import pallas as pl
from jax.experimental.pallas import tpu as pltpu
from jax.experimental.pallas import tpu_sc as plsc

NC = 2
NS = 16
NW = NC * NS


@functools.lru_cache(maxsize=None)
def _make_gather(n, v, d, chunk):
    num_chunks = n // (NW * chunk)
    b_per_w = n // NW
    mesh = plsc.VectorSubcoreMesh(core_axis_name="c", subcore_axis_name="s")

    @functools.partial(
        pl.kernel,
        mesh=mesh,
        out_type=jax.ShapeDtypeStruct((n, d), jnp.float32),
        scratch_types=[
            pltpu.VMEM((chunk,), jnp.int32),
            pltpu.VMEM((chunk, d), jnp.float32),
            pltpu.SemaphoreType.DMA,
        ],
    )
    def k(table_hbm, idx_hbm, out_hbm, idx_v, rows_v, sem):
        wid = lax.axis_index("s") * NC + lax.axis_index("c")
        base = wid * b_per_w

        def body(g, carry):
            off = base + g * chunk
            pltpu.sync_copy(idx_hbm.at[pl.ds(off, chunk)], idx_v)
            pltpu.async_copy(table_hbm.at[idx_v], rows_v, sem).wait()
            pltpu.sync_copy(rows_v, out_hbm.at[pl.ds(off, chunk)])
            return carry

        lax.fori_loop(0, num_chunks, body, 0)

    return k


def kernel(item_inputs, table):
    b, h = item_inputs.shape
    v, d = table.shape
    n = b * h
    idx = item_inputs.reshape(n).astype(jnp.int32)
    out = _make_gather(n, v, d, 1024)(table, idx)
    return out.reshape(b, h, d)


# speedup vs baseline: 1.0951x; 1.0951x over previous
import functools

import jax
import jax.numpy as jnp
from jax import lax
from jax.experimental import pallas as pl
from jax.experimental.pallas import tpu as pltpu
from jax.experimental.pallas import tpu_sc as plsc

NC = 2
NS = 16
NW = NC * NS


@functools.lru_cache(maxsize=None)
def _make_gather(n, v, d, chunk):
    num_chunks = n // (NW * chunk)
    b_per_w = n // NW
    mesh = plsc.VectorSubcoreMesh(core_axis_name="c", subcore_axis_name="s")

    @functools.partial(
        pl.kernel,
        mesh=mesh,
        out_type=jax.ShapeDtypeStruct((n, d), jnp.float32),
        scratch_types=[
            pltpu.VMEM((chunk,), jnp.int32),
            pltpu.VMEM((chunk, d), jnp.float32),
            pltpu.SemaphoreType.DMA,
        ],
        compiler_params=pltpu.CompilerParams(use_tc_tiling_on_sc=False),
    )
    def k(table_hbm, idx_hbm, out_hbm, idx_v, rows_v, sem):
        wid = lax.axis_index("s") * NC + lax.axis_index("c")
        base = wid * b_per_w

        def body(g, carry):
            off = base + g * chunk
            pltpu.sync_copy(idx_hbm.at[pl.ds(off, chunk)], idx_v)
            pltpu.async_copy(table_hbm.at[idx_v], rows_v, sem).wait()
            pltpu.sync_copy(rows_v, out_hbm.at[pl.ds(off, chunk)])
            return carry

        lax.fori_loop(0, num_chunks, body, 0)

    return k


def kernel(item_inputs, table):
    b, h = item_inputs.shape
    v, d = table.shape
    n = b * h
    idx = item_inputs.reshape(n).astype(jnp.int32)
    out = _make_gather(n, v, d, 1024)(table, idx)
    return out.reshape(b, h, d)


# speedup vs baseline: 1.1132x; 1.0166x over previous
import functools

import jax
import jax.numpy as jnp
from jax import lax
from jax.experimental import pallas as pl
from jax.experimental.pallas import tpu as pltpu
from jax.experimental.pallas import tpu_sc as plsc

NC = 2
NS = 16
NW = NC * NS


@functools.lru_cache(maxsize=None)
def _make_gather(n, v, d, chunk, nbuf):
    b_per_w = n // NW
    num_chunks = b_per_w // chunk
    n_outer = num_chunks // nbuf
    assert num_chunks % nbuf == 0 and n % (NW * chunk) == 0
    mesh = plsc.VectorSubcoreMesh(core_axis_name="c", subcore_axis_name="s")

    @functools.partial(
        pl.kernel,
        mesh=mesh,
        out_type=jax.ShapeDtypeStruct((n, d), jnp.float32),
        scratch_types=(
            [pltpu.VMEM((b_per_w,), jnp.int32)]
            + [pltpu.VMEM((chunk, d), jnp.float32) for _ in range(nbuf)]
            + [pltpu.SemaphoreType.DMA for _ in range(2 * nbuf)]
        ),
        compiler_params=pltpu.CompilerParams(use_tc_tiling_on_sc=False),
    )
    def k(table_hbm, idx_hbm, out_hbm, idx_v, *bufs_sems):
        bufs = bufs_sems[:nbuf]
        gsems = bufs_sems[nbuf:2 * nbuf]
        ssems = bufs_sems[2 * nbuf:]
        wid = lax.axis_index("s") * NC + lax.axis_index("c")
        base = wid * b_per_w

        pltpu.sync_copy(idx_hbm.at[pl.ds(base, b_per_w)], idx_v)

        def fire_gather(g, b):
            pltpu.async_copy(
                table_hbm.at[idx_v.at[pl.ds(g * chunk, chunk)]], bufs[b], gsems[b])

        for b in range(nbuf):
            fire_gather(b, b)

        def body(t, carry):
            for b in range(nbuf):
                g = t * nbuf + b
                pltpu.make_async_copy(
                    table_hbm.at[idx_v.at[pl.ds(0, chunk)]], bufs[b], gsems[b]).wait()
                pltpu.async_copy(
                    bufs[b], out_hbm.at[pl.ds(base + g * chunk, chunk)], ssems[b])

                @pl.when(t < n_outer - 1)
                def _():
                    pltpu.make_async_copy(
                        bufs[b], out_hbm.at[pl.ds(base, chunk)], ssems[b]).wait()
                    fire_gather(g + nbuf, b)
            return carry

        lax.fori_loop(0, n_outer, body, 0)
        for b in range(nbuf):
            pltpu.make_async_copy(
                bufs[b], out_hbm.at[pl.ds(base, chunk)], ssems[b]).wait()

    return k


def kernel(item_inputs, table):
    b, h = item_inputs.shape
    v, d = table.shape
    n = b * h
    idx = item_inputs.reshape(n).astype(jnp.int32)
    out = _make_gather(n, v, d, 640, 4)(table, idx)
    return out.reshape(b, h, d)


# speedup vs baseline: 1.9450x; 1.7472x over previous
import functools

import jax
import jax.numpy as jnp
from jax import lax
from jax.experimental import pallas as pl
from jax.experimental.pallas import tpu as pltpu
from jax.experimental.pallas import tpu_sc as plsc

NC = 2
NS = 16
NW = NC * NS


@functools.lru_cache(maxsize=None)
def _make_gather(n, v, d, chunk, nbuf):
    b_per_w = n // NW
    num_chunks = b_per_w // chunk
    n_outer = num_chunks // nbuf
    assert num_chunks % nbuf == 0 and n % (NW * chunk) == 0
    mesh = plsc.VectorSubcoreMesh(core_axis_name="c", subcore_axis_name="s")

    @functools.partial(
        pl.kernel,
        mesh=mesh,
        out_type=jax.ShapeDtypeStruct((n, d), jnp.float32),
        scratch_types=(
            [pltpu.VMEM((b_per_w,), jnp.int32)]
            + [pltpu.VMEM((chunk, d), jnp.float32) for _ in range(nbuf)]
            + [pltpu.SemaphoreType.DMA for _ in range(2 * nbuf)]
        ),
        compiler_params=pltpu.CompilerParams(use_tc_tiling_on_sc=False),
    )
    def k(table_hbm, idx_hbm, out_hbm, idx_v, *bufs_sems):
        bufs = bufs_sems[:nbuf]
        gsems = bufs_sems[nbuf:2 * nbuf]
        ssems = bufs_sems[2 * nbuf:]
        wid = lax.axis_index("s") * NC + lax.axis_index("c")
        base = wid * b_per_w

        pltpu.sync_copy(idx_hbm.at[pl.ds(base, b_per_w)], idx_v)

        def fire_gather(g, b):
            pltpu.async_copy(
                table_hbm.at[idx_v.at[pl.ds(g * chunk, chunk)]], bufs[b], gsems[b])

        for b in range(nbuf):
            fire_gather(b, b)

        def body(t, carry):
            for b in range(nbuf):
                g = t * nbuf + b
                pltpu.make_async_copy(
                    table_hbm.at[idx_v.at[pl.ds(0, chunk)]], bufs[b], gsems[b]).wait()
                pltpu.async_copy(
                    bufs[b], out_hbm.at[pl.ds(base + g * chunk, chunk)], ssems[b])

                @pl.when(t < n_outer - 1)
                def _():
                    pltpu.make_async_copy(
                        bufs[b], out_hbm.at[pl.ds(base, chunk)], ssems[b]).wait()
                    fire_gather(g + nbuf, b)
            return carry

        lax.fori_loop(0, n_outer, body, 0)
        for b in range(nbuf):
            pltpu.make_async_copy(
                bufs[b], out_hbm.at[pl.ds(base, chunk)], ssems[b]).wait()

    return k


def kernel(item_inputs, table):
    b, h = item_inputs.shape
    v, d = table.shape
    n = b * h
    idx = item_inputs.T.reshape(n).astype(jnp.int32)
    out = _make_gather(n, v, d, 640, 4)(table, idx)
    return out.reshape(h, b, d).transpose(1, 0, 2)
